# baseline (speedup 1.0000x reference)
import jax
import jax.numpy as jnp
from jax import lax
from jax.experimental import pallas as pl
from jax.experimental.pallas import tpu as pltpu
from jax.experimental.pallas import tpu_sc as plsc

B = 4096
L = 50
EMB = 300
EMBP = 304
ACCC = 304
HID = 300
NCLS = 1000

NC = 2
NS = 16
NW = NC * NS

EX_PER_W = B // NW
HALF = EX_PER_W // 2


MAINC = 256
TAILC = 128


def _sc_pool_body(idxt_hbm, emb_hbm, tail_hbm, zeros_hbm, out_hbm,
                  idx_v, acc_v, main0_v, main1_v, tail0_v, tail1_v, sem0, sem1):
    c = lax.axis_index("c")
    s = lax.axis_index("s")
    wid = s * NC + c

    pltpu.sync_copy(idxt_hbm.at[:, pl.ds(wid * EX_PER_W, EX_PER_W)], idx_v)
    pltpu.sync_copy(zeros_hbm, acc_v)

    def start_gather(g, h, mbuf, tbuf, sem):
        isl = idx_v.at[g, pl.ds(h * HALF, HALF)]
        pltpu.async_copy(emb_hbm.at[isl, pl.ds(0, MAINC)], mbuf, sem)
        pltpu.async_copy(tail_hbm.at[isl], tbuf, sem)

    def wait_gather(mbuf, tbuf, sem):
        isl = idx_v.at[0, pl.ds(0, HALF)]
        pltpu.make_async_copy(emb_hbm.at[isl, pl.ds(0, MAINC)], mbuf, sem).wait()
        pltpu.make_async_copy(tail_hbm.at[isl], tbuf, sem).wait()

    def accumulate(mbuf, tbuf, row_base):
        @plsc.parallel_loop(0, HALF, step=1, unroll=8)
        def row_body(r):
            msrc = mbuf.at[r]
            tsrc = tbuf.at[r]
            dst = acc_v.at[row_base + r]
            for cc in range(MAINC // 16):
                plsc.addupdate(dst.at[pl.ds(cc * 16, 16)],
                               msrc[pl.ds(cc * 16, 16)])
            for cc in range((ACCC - MAINC) // 16):
                plsc.addupdate(dst.at[pl.ds(MAINC + cc * 16, 16)],
                               tsrc[pl.ds(cc * 16, 16)])

    start_gather(0, 0, main0_v, tail0_v, sem0)

    def g_body(g, carry):
        start_gather(g, 1, main1_v, tail1_v, sem1)
        wait_gather(main0_v, tail0_v, sem0)
        accumulate(main0_v, tail0_v, 0)

        @pl.when(g < L - 1)
        def _():
            start_gather(g + 1, 0, main0_v, tail0_v, sem0)

        wait_gather(main1_v, tail1_v, sem1)
        accumulate(main1_v, tail1_v, HALF)
        return carry

    lax.fori_loop(0, L, g_body, 0)

    pltpu.sync_copy(acc_v, out_hbm.at[pl.ds(wid * EX_PER_W, EX_PER_W)])


def _sc_pool(idx_t, emb, emb_tail, zeros):
    fn = pl.kernel(
        _sc_pool_body,
        out_type=jax.ShapeDtypeStruct((B, EMBP), jnp.float32),
        mesh=plsc.VectorSubcoreMesh(core_axis_name="c", subcore_axis_name="s",
                                    num_cores=NC, num_subcores=NS),
        scratch_types=[
            pltpu.VMEM((L, EX_PER_W), jnp.int32),
            pltpu.VMEM((EX_PER_W, EMBP), jnp.float32),
            pltpu.VMEM((HALF, MAINC), jnp.float32),
            pltpu.VMEM((HALF, MAINC), jnp.float32),
            pltpu.VMEM((HALF, TAILC), jnp.float32),
            pltpu.VMEM((HALF, TAILC), jnp.float32),
            pltpu.SemaphoreType.DMA,
            pltpu.SemaphoreType.DMA,
        ],
    )
    return fn(idx_t, emb, emb_tail, zeros)


def _mlp_body(x_ref, len_ref, w1_ref, b1_ref, w2_ref, b2_ref, out_ref):
    x = x_ref[...] / len_ref[...]
    h = jnp.dot(x, w1_ref[...], preferred_element_type=jnp.float32) + b1_ref[...]
    h = jnp.maximum(h, 0.0)
    out_ref[...] = jnp.dot(h, w2_ref[...], preferred_element_type=jnp.float32) + b2_ref[...]


def _mlp(pooled, text_len, W1p, b1, W2, b2):
    BLK = 1024
    grid = (B // BLK,)
    return pl.pallas_call(
        _mlp_body,
        grid=grid,
        in_specs=[
            pl.BlockSpec((BLK, EMBP), lambda i: (i, 0)),
            pl.BlockSpec((BLK, 1), lambda i: (i, 0)),
            pl.BlockSpec((EMBP, HID), lambda i: (0, 0)),
            pl.BlockSpec((1, HID), lambda i: (0, 0)),
            pl.BlockSpec((HID, NCLS), lambda i: (0, 0)),
            pl.BlockSpec((1, NCLS), lambda i: (0, 0)),
        ],
        out_specs=pl.BlockSpec((BLK, NCLS), lambda i: (i, 0)),
        out_shape=jax.ShapeDtypeStruct((B, NCLS), jnp.float32),
    )(pooled, text_len, W1p, b1, W2, b2)


def kernel(input_text, text_len, emb, W1, b1, W2, b2):
    idx_t = input_text.T
    emb_tail = jnp.pad(emb[:, MAINC:], ((0, 0), (0, TAILC - (EMB - MAINC))))
    zeros = jnp.zeros((EX_PER_W, EMBP), jnp.float32)
    pooled = _sc_pool(idx_t, emb, emb_tail, zeros)
    W1p = jnp.pad(W1, ((0, EMBP - EMB), (0, 0)))
    return _mlp(pooled, text_len.reshape(B, 1), W1p, b1.reshape(1, HID),
                W2, b2.reshape(1, NCLS))
